# baseline (speedup 1.0000x reference)
import functools

import jax
import jax.numpy as jnp
from jax import lax
from jax.experimental import pallas as pl
from jax.experimental.pallas import tpu as pltpu
from jax.experimental.pallas import tpu_sc as plsc

HIDDEN = 2048
NUM_EXPERTS = 64
TOKENS = 16384
BT = 2048

NC = 2
NS = 16
L = 16
NW = NC * NS
TPW = TOKENS // NW
GROUPS = TPW // L


def _logits_block(x_ref, w_ref, out_ref):
    out_ref[...] = jax.lax.dot_general(
        w_ref[...], x_ref[...],
        dimension_numbers=(((1,), (1,)), ((), ())),
        preferred_element_type=jnp.float32,
    )


def _tc_logits(x, weight):
    grid = (TOKENS // BT,)
    return pl.pallas_call(
        _logits_block,
        grid=grid,
        in_specs=[
            pl.BlockSpec((BT, HIDDEN), lambda i: (i, 0)),
            pl.BlockSpec((NUM_EXPERTS, HIDDEN), lambda i: (0, 0)),
        ],
        out_specs=pl.BlockSpec((NUM_EXPERTS, BT), lambda i: (0, i)),
        out_shape=jax.ShapeDtypeStruct((NUM_EXPERTS, TOKENS), jnp.float32),
    )(x, weight)


def _sc_select(logits):
    mesh = plsc.VectorSubcoreMesh(core_axis_name="c", subcore_axis_name="s")

    @functools.partial(
        pl.kernel,
        mesh=mesh,
        out_type=[
            jax.ShapeDtypeStruct((2, TOKENS), jnp.float32),
            jax.ShapeDtypeStruct((2, TOKENS), jnp.int32),
        ],
        scratch_types=[
            pltpu.VMEM((NUM_EXPERTS, TPW), jnp.float32),
            pltpu.VMEM((2, TPW), jnp.float32),
            pltpu.VMEM((2, TPW), jnp.int32),
        ],
    )
    def sc_kernel(logits_hbm, outw_hbm, outi_hbm, lbuf, wbuf, ibuf):
        wid = lax.axis_index("s") * NC + lax.axis_index("c")
        base = wid * TPW
        pltpu.sync_copy(logits_hbm.at[:, pl.ds(base, TPW)], lbuf)

        def group_body(g, carry):
            off = g * L
            neg = jnp.full((L,), -jnp.inf, dtype=jnp.float32)
            m0, m1 = neg, neg
            zero = jnp.zeros((L,), dtype=jnp.int32)
            i0, i1 = zero, zero
            for e in range(NUM_EXPERTS):
                v = lbuf[e, pl.ds(off, L)]
                evec = jnp.full((L,), e, dtype=jnp.int32)
                gt0 = v > m0
                gt1 = v > m1
                m1 = jnp.where(gt0, m0, jnp.where(gt1, v, m1))
                i1 = jnp.where(gt0, i0, jnp.where(gt1, evec, i1))
                m0 = jnp.where(gt0, v, m0)
                i0 = jnp.where(gt0, evec, i0)
            e1 = jnp.exp(m1 - m0)
            denom = 1.0 + e1
            wbuf[0, pl.ds(off, L)] = 1.0 / denom
            wbuf[1, pl.ds(off, L)] = e1 / denom
            ibuf[0, pl.ds(off, L)] = i0
            ibuf[1, pl.ds(off, L)] = i1
            return carry

        lax.fori_loop(0, GROUPS, group_body, 0)
        pltpu.sync_copy(wbuf, outw_hbm.at[:, pl.ds(base, TPW)])
        pltpu.sync_copy(ibuf, outi_hbm.at[:, pl.ds(base, TPW)])

    return sc_kernel(logits)


@jax.jit
def kernel(x, weight):
    logits = _tc_logits(x, weight)
    outw, outi = _sc_select(logits)
    return (outw.T, outi.T)
